# baseline (speedup 1.0000x reference)
import jax
import jax.numpy as jnp
from jax.experimental import pallas as pl
from jax.experimental.pallas import tpu as pltpu

_BN = 2048


def _add_body(x_ref, pos_ref, o_ref):
    o_ref[...] = x_ref[...] + pos_ref[...]


@jax.jit
def kernel(x, token_embedding):
    B, N, D = x.shape
    nb = N // _BN
    out = pl.pallas_call(
        _add_body,
        grid=(nb, B),
        in_specs=[
            pl.BlockSpec((_BN, D), lambda n, b: (b * nb + n, 0)),
            pl.BlockSpec((_BN, D), lambda n, b: (n, 0)),
        ],
        out_specs=pl.BlockSpec((_BN, D), lambda n, b: (b * nb + n, 0)),
        out_shape=jax.ShapeDtypeStruct((B * N, D), x.dtype),
        compiler_params=pltpu.CompilerParams(
            dimension_semantics=("parallel", "arbitrary"),
        ),
    )(x.reshape(B * N, D), token_embedding)
    return out.reshape(B, N, D)


# speedup vs baseline: 1.0023x; 1.0023x over previous
import jax
import jax.numpy as jnp
from jax.experimental import pallas as pl

_BN = 2048


def _add_body(x_ref, pos_ref, o_ref):
    o_ref[0] = x_ref[0] + pos_ref[...]


@jax.jit
def kernel(x, token_embedding):
    B, N, D = x.shape
    return pl.pallas_call(
        _add_body,
        grid=(N // _BN, B),
        in_specs=[
            pl.BlockSpec((1, _BN, D), lambda n, b: (b, n, 0)),
            pl.BlockSpec((_BN, D), lambda n, b: (n, 0)),
        ],
        out_specs=pl.BlockSpec((1, _BN, D), lambda n, b: (b, n, 0)),
        out_shape=jax.ShapeDtypeStruct((B, N, D), x.dtype),
    )(x, token_embedding)


# speedup vs baseline: 1.0208x; 1.0185x over previous
import jax
import jax.numpy as jnp
from jax.experimental import pallas as pl
from jax.experimental.pallas import tpu as pltpu

_CH = 1024
_DEPTH = 3


def _body(x_hbm, pos_hbm, o_hbm, x_buf, o_buf, pos_vmem, in_sems, out_sems, pos_sems):
    R, D = x_hbm.shape
    NP = pos_hbm.shape[0] // _CH
    NCH = R // _CH

    def in_cp(c, k):
        return pltpu.make_async_copy(
            x_hbm.at[pl.ds(c * _CH, _CH)], x_buf.at[k], in_sems.at[k])

    def out_cp(c, k):
        return pltpu.make_async_copy(
            o_buf.at[k], o_hbm.at[pl.ds(c * _CH, _CH)], out_sems.at[k])

    def pos_cp(j):
        return pltpu.make_async_copy(
            pos_hbm.at[pl.ds(j * _CH, _CH)],
            pos_vmem.at[pl.ds(j * _CH, _CH)], pos_sems.at[j])

    pos_cp(0).start()
    for k in range(_DEPTH):
        in_cp(k, k).start()
    for j in range(1, NP):
        pos_cp(j).start()

    for c in range(NCH):
        k = c % _DEPTH
        j = c % NP
        if c < NP:
            pos_cp(j).wait()
        in_cp(c, k).wait()
        if c >= _DEPTH:
            out_cp(c - _DEPTH, k).wait()
        o_buf[k] = x_buf[k] + pos_vmem[pl.ds(j * _CH, _CH), :]
        out_cp(c, k).start()
        if c + _DEPTH < NCH:
            in_cp(c + _DEPTH, k).start()

    for c in range(NCH - _DEPTH, NCH):
        out_cp(c, c % _DEPTH).wait()


@jax.jit
def kernel(x, token_embedding):
    B, N, D = x.shape
    out = pl.pallas_call(
        _body,
        in_specs=[
            pl.BlockSpec(memory_space=pltpu.HBM),
            pl.BlockSpec(memory_space=pltpu.HBM),
        ],
        out_specs=pl.BlockSpec(memory_space=pltpu.HBM),
        out_shape=jax.ShapeDtypeStruct((B * N, D), x.dtype),
        scratch_shapes=[
            pltpu.VMEM((_DEPTH, _CH, D), x.dtype),
            pltpu.VMEM((_DEPTH, _CH, D), x.dtype),
            pltpu.VMEM((N, D), x.dtype),
            pltpu.SemaphoreType.DMA((_DEPTH,)),
            pltpu.SemaphoreType.DMA((_DEPTH,)),
            pltpu.SemaphoreType.DMA((N // _CH,)),
        ],
    )(x.reshape(B * N, D), token_embedding)
    return out.reshape(B, N, D)
